# baseline (speedup 1.0000x reference)
import functools

import jax
import jax.numpy as jnp
from jax import lax
from jax.experimental import pallas as pl
from jax.experimental.pallas import tpu as pltpu
from jax.experimental.pallas import tpu_sc as plsc

KP_EXTENT = 2.0
EPS = 1e-5


def _sc_gather(xcat, inds_flat, chunk=400):
    n_rows, d = xcat.shape
    e = inds_flat.shape[0]
    info = plsc.get_sparse_core_info()
    nw = info.num_cores * info.num_subcores
    per_w = e // nw
    assert e % nw == 0 and per_w % chunk == 0 and chunk % 8 == 0
    n_ch = per_w // chunk
    mesh = plsc.VectorSubcoreMesh(core_axis_name="c", subcore_axis_name="s")

    @functools.partial(
        pl.kernel,
        mesh=mesh,
        out_type=jax.ShapeDtypeStruct((e, d), jnp.float32),
        scratch_types=[
            pltpu.VMEM((chunk,), jnp.int32),
            pltpu.VMEM((chunk, d), jnp.float32),
            pltpu.SemaphoreType.DMA,
        ],
        compiler_params=pltpu.CompilerParams(use_tc_tiling_on_sc=False),
    )
    def gather_k(xcat_hbm, idx_hbm, out_hbm, idx_v, rows_v, sem):
        wid = lax.axis_index("s") * info.num_cores + lax.axis_index("c")
        base0 = wid * per_w

        def body(i, carry):
            base = base0 + i * chunk
            pltpu.sync_copy(idx_hbm.at[pl.ds(base, chunk)], idx_v)
            pltpu.async_copy(xcat_hbm.at[idx_v], rows_v, sem).wait()
            pltpu.sync_copy(rows_v, out_hbm.at[pl.ds(base, chunk)])
            return carry

        lax.fori_loop(0, n_ch, body, 0)

    return gather_k(xcat, inds_flat)


def _p1_body(k1, k2, c, o2, n_total, g_ref, q_ref, kp_ref, wm_ref, wd_ref,
             x1r_ref, x2r_ref, st_ref):
    g = g_ref[...]
    feat = g[:, :, :c]
    xyz = g[:, :, c:c + 3]
    q = q_ref[...]
    nb = xyz - q[:, None, :]

    b = g.shape[0]
    acc1 = jnp.zeros((b, o2), jnp.float32)
    acc2 = jnp.zeros((b, o2), jnp.float32)
    inv_ext = 1.0 / KP_EXTENT
    for k in range(k1 + k2):
        kp_k = kp_ref[k:k + 1, :].reshape(1, 1, 3)
        d2 = nb - kp_k
        sqk = jnp.sum(d2 * d2, axis=2)
        wk = jnp.maximum(1.0 - jnp.sqrt(sqk + 1e-12) * inv_ext, 0.0)
        wfk = jnp.sum(wk[:, :, None] * feat, axis=1)
        if k < k1:
            acc1 = acc1 + jnp.dot(wfk, wm_ref[k],
                                  preferred_element_type=jnp.float32)
        else:
            acc2 = acc2 + jnp.dot(wfk, wd_ref[k - k1],
                                  preferred_element_type=jnp.float32)

    x1r_ref[...] = acc1
    x2r_ref[...] = acc2

    s1 = jnp.sum(acc1, axis=0, keepdims=True)
    q1 = jnp.sum(acc1 * acc1, axis=0, keepdims=True)
    s2 = jnp.sum(acc2, axis=0, keepdims=True)
    q2 = jnp.sum(acc2 * acc2, axis=0, keepdims=True)
    blk = jnp.concatenate(
        [s1, q1, s2, q2, jnp.zeros((4, o2), jnp.float32)], axis=0)

    @pl.when(pl.program_id(0) == 0)
    def _():
        st_ref[...] = jnp.zeros_like(st_ref)

    st_ref[...] += blk


def _p2_body(n_total, x1r_ref, x2r_ref, st_ref, g1_ref, b1_ref, g2_ref,
             b2_ref, wm_ref, wft_ref, wfb_ref, outr_ref, stf_ref):
    st = st_ref[...]
    inv_n = 1.0 / n_total
    m1 = st[0:1, :] * inv_n
    v1 = st[1:2, :] * inv_n - m1 * m1
    a1 = g1_ref[...] / jnp.sqrt(v1 + EPS)
    c1 = b1_ref[...] - a1 * m1
    m2 = st[2:3, :] * inv_n
    v2 = st[3:4, :] * inv_n - m2 * m2
    a2 = g2_ref[...] / jnp.sqrt(v2 + EPS)
    c2 = b2_ref[...] - a2 * m2

    x1 = x1r_ref[...] * a1 + c1
    x2 = x2r_ref[...] * a2 + c2
    y2 = jnp.dot(x1 + x2, wm_ref[...], preferred_element_type=jnp.float32)
    outr = (jnp.dot(x1, wft_ref[...], preferred_element_type=jnp.float32)
            + jnp.dot(y2, wfb_ref[...], preferred_element_type=jnp.float32))
    outr_ref[...] = outr

    s = jnp.sum(outr, axis=0, keepdims=True)
    qq = jnp.sum(outr * outr, axis=0, keepdims=True)
    o = outr.shape[1]
    blk = jnp.concatenate([s, qq, jnp.zeros((6, o), jnp.float32)], axis=0)

    @pl.when(pl.program_id(0) == 0)
    def _():
        stf_ref[...] = jnp.zeros_like(stf_ref)

    stf_ref[...] += blk


def _p3_body(n_total, outr_ref, stf_ref, gf_ref, bf_ref, out_ref):
    st = stf_ref[...]
    inv_n = 1.0 / n_total
    m = st[0:1, :] * inv_n
    v = st[1:2, :] * inv_n - m * m
    a = gf_ref[...] / jnp.sqrt(v + EPS)
    c = bf_ref[...] - a * m
    out_ref[...] = jnp.maximum(outr_ref[...] * a + c, 0.0)


def kernel(q_pts, s_pts, neighb_inds, x, stack_lengths_post, KP_mini, W_mini,
           gamma1, beta1, KP_mid, W_mid, gamma2, beta2, W_midmini, W_final,
           gamma_f, beta_f):
    n, c = x.shape
    h = neighb_inds.shape[1]
    k1 = KP_mini.shape[0]
    k2 = KP_mid.shape[0]
    o2 = W_mini.shape[2]
    o = W_final.shape[1]
    d = ((c + 3 + 15) // 16) * 16

    xcat = jnp.concatenate(
        [x, s_pts, jnp.zeros((n, d - c - 3), jnp.float32)], axis=1)
    inds_flat = neighb_inds.astype(jnp.int32).reshape(-1)
    gathered = _sc_gather(xcat, inds_flat)
    g3 = gathered.reshape(n, h, d)

    kp_all = jnp.concatenate([KP_mini, KP_mid], axis=0)

    b = 200
    grid = (n // b,)
    x1r, x2r, st = pl.pallas_call(
        functools.partial(_p1_body, k1, k2, c, o2, n),
        grid=grid,
        in_specs=[
            pl.BlockSpec((b, h, d), lambda i: (i, 0, 0)),
            pl.BlockSpec((b, 3), lambda i: (i, 0)),
            pl.BlockSpec((k1 + k2, 3), lambda i: (0, 0)),
            pl.BlockSpec((k1, c, o2), lambda i: (0, 0, 0)),
            pl.BlockSpec((k2, c, o2), lambda i: (0, 0, 0)),
        ],
        out_specs=[
            pl.BlockSpec((b, o2), lambda i: (i, 0)),
            pl.BlockSpec((b, o2), lambda i: (i, 0)),
            pl.BlockSpec((8, o2), lambda i: (0, 0)),
        ],
        out_shape=[
            jax.ShapeDtypeStruct((n, o2), jnp.float32),
            jax.ShapeDtypeStruct((n, o2), jnp.float32),
            jax.ShapeDtypeStruct((8, o2), jnp.float32),
        ],
    )(g3, q_pts, kp_all, W_mini, W_mid)

    wf_top = W_final[:o2, :]
    wf_bot = W_final[o2:, :]
    outr, stf = pl.pallas_call(
        functools.partial(_p2_body, n),
        grid=grid,
        in_specs=[
            pl.BlockSpec((b, o2), lambda i: (i, 0)),
            pl.BlockSpec((b, o2), lambda i: (i, 0)),
            pl.BlockSpec((8, o2), lambda i: (0, 0)),
            pl.BlockSpec((1, o2), lambda i: (0, 0)),
            pl.BlockSpec((1, o2), lambda i: (0, 0)),
            pl.BlockSpec((1, o2), lambda i: (0, 0)),
            pl.BlockSpec((1, o2), lambda i: (0, 0)),
            pl.BlockSpec((o2, o2), lambda i: (0, 0)),
            pl.BlockSpec((o2, o), lambda i: (0, 0)),
            pl.BlockSpec((o2, o), lambda i: (0, 0)),
        ],
        out_specs=[
            pl.BlockSpec((b, o), lambda i: (i, 0)),
            pl.BlockSpec((8, o), lambda i: (0, 0)),
        ],
        out_shape=[
            jax.ShapeDtypeStruct((n, o), jnp.float32),
            jax.ShapeDtypeStruct((8, o), jnp.float32),
        ],
    )(x1r, x2r, st, gamma1.reshape(1, o2), beta1.reshape(1, o2),
      gamma2.reshape(1, o2), beta2.reshape(1, o2), W_midmini, wf_top, wf_bot)

    out = pl.pallas_call(
        functools.partial(_p3_body, n),
        grid=grid,
        in_specs=[
            pl.BlockSpec((b, o), lambda i: (i, 0)),
            pl.BlockSpec((8, o), lambda i: (0, 0)),
            pl.BlockSpec((1, o), lambda i: (0, 0)),
            pl.BlockSpec((1, o), lambda i: (0, 0)),
        ],
        out_specs=pl.BlockSpec((b, o), lambda i: (i, 0)),
        out_shape=jax.ShapeDtypeStruct((n, o), jnp.float32),
    )(outr, stf, gamma_f.reshape(1, o), beta_f.reshape(1, o))

    return out


# speedup vs baseline: 1.6870x; 1.6870x over previous
import functools

import jax
import jax.numpy as jnp
from jax import lax
from jax.experimental import pallas as pl
from jax.experimental.pallas import tpu as pltpu
from jax.experimental.pallas import tpu_sc as plsc

KP_EXTENT = 2.0
EPS = 1e-5


def _sc_gather(x, sx, sy, sz, inds_flat, chunk=400):
    n, c = x.shape
    e = inds_flat.shape[0]
    info = plsc.get_sparse_core_info()
    nw = info.num_cores * info.num_subcores
    per_w = e // nw
    assert e % nw == 0 and per_w % chunk == 0 and chunk % 16 == 0
    n_ch = per_w // chunk
    n_sub = chunk // 16
    mesh = plsc.VectorSubcoreMesh(core_axis_name="c", subcore_axis_name="s")

    @functools.partial(
        pl.kernel,
        mesh=mesh,
        out_type=(
            jax.ShapeDtypeStruct((e, c), jnp.float32),
            jax.ShapeDtypeStruct((e,), jnp.float32),
            jax.ShapeDtypeStruct((e,), jnp.float32),
            jax.ShapeDtypeStruct((e,), jnp.float32),
        ),
        scratch_types=[
            pltpu.VMEM((chunk,), jnp.int32),
            pltpu.VMEM((chunk, c), jnp.float32),
            pltpu.VMEM((n,), jnp.float32),
            pltpu.VMEM((n,), jnp.float32),
            pltpu.VMEM((n,), jnp.float32),
            pltpu.VMEM((chunk,), jnp.float32),
            pltpu.VMEM((chunk,), jnp.float32),
            pltpu.VMEM((chunk,), jnp.float32),
            pltpu.SemaphoreType.DMA,
        ],
        compiler_params=pltpu.CompilerParams(needs_layout_passes=False),
    )
    def gather_k(x_hbm, sx_hbm, sy_hbm, sz_hbm, idx_hbm,
                 feat_hbm, ox_hbm, oy_hbm, oz_hbm,
                 idx_v, rows_v, sxv, syv, szv, gxv, gyv, gzv, sem):
        wid = lax.axis_index("s") * info.num_cores + lax.axis_index("c")
        base0 = wid * per_w
        pltpu.sync_copy(sx_hbm, sxv)
        pltpu.sync_copy(sy_hbm, syv)
        pltpu.sync_copy(sz_hbm, szv)

        def chunk_body(i, carry):
            base = base0 + i * chunk
            pltpu.sync_copy(idx_hbm.at[pl.ds(base, chunk)], idx_v)
            pltpu.async_copy(x_hbm.at[idx_v], rows_v, sem).wait()

            def sub_body(j, carry2):
                off = j * 16
                iv = idx_v[pl.ds(off, 16)]
                gxv[pl.ds(off, 16)] = plsc.load_gather(sxv, [iv])
                gyv[pl.ds(off, 16)] = plsc.load_gather(syv, [iv])
                gzv[pl.ds(off, 16)] = plsc.load_gather(szv, [iv])
                return carry2

            lax.fori_loop(0, n_sub, sub_body, 0)
            pltpu.sync_copy(rows_v, feat_hbm.at[pl.ds(base, chunk)])
            pltpu.sync_copy(gxv, ox_hbm.at[pl.ds(base, chunk)])
            pltpu.sync_copy(gyv, oy_hbm.at[pl.ds(base, chunk)])
            pltpu.sync_copy(gzv, oz_hbm.at[pl.ds(base, chunk)])
            return carry

        lax.fori_loop(0, n_ch, chunk_body, 0)

    return gather_k(x, sx, sy, sz, inds_flat)


def _p1_body(k1, k2, c, o2, g_ref, nbx_ref, nby_ref, nbz_ref, q_ref, kp_ref,
             wm_ref, wd_ref, x1r_ref, x2r_ref, st_ref, w3_ref, wf_ref):
    b = q_ref.shape[0]
    h = nbx_ref.shape[1]
    kk = k1 + k2
    nbx = nbx_ref[...] - q_ref[:, 0:1]
    nby = nby_ref[...] - q_ref[:, 1:2]
    nbz = nbz_ref[...] - q_ref[:, 2:3]

    inv_ext = 1.0 / KP_EXTENT
    wks = []
    for k in range(kk):
        dx = nbx - kp_ref[k:k + 1, 0:1]
        dy = nby - kp_ref[k:k + 1, 1:2]
        dz = nbz - kp_ref[k:k + 1, 2:3]
        sqk = dx * dx + dy * dy + dz * dz
        wk = jnp.maximum(1.0 - jnp.sqrt(sqk + 1e-12) * inv_ext, 0.0)
        wks.append(wk)
    w3_ref[...] = jnp.stack(wks, axis=1)

    def mm_body(bb, carry):
        w_b = w3_ref[pl.ds(bb, 1)][0]
        f_b = g_ref[pl.ds(bb, 1)][0]
        wf_ref[pl.ds(bb, 1)] = jnp.dot(
            w_b, f_b, preferred_element_type=jnp.float32)[None]
        return carry

    lax.fori_loop(0, b, mm_body, 0)

    wf = wf_ref[...]
    acc1 = jnp.zeros((b, o2), jnp.float32)
    acc2 = jnp.zeros((b, o2), jnp.float32)
    for k in range(kk):
        wfk = wf[:, k, :]
        if k < k1:
            acc1 = acc1 + jnp.dot(wfk, wm_ref[k],
                                  preferred_element_type=jnp.float32)
        else:
            acc2 = acc2 + jnp.dot(wfk, wd_ref[k - k1],
                                  preferred_element_type=jnp.float32)

    x1r_ref[...] = acc1
    x2r_ref[...] = acc2

    s1 = jnp.sum(acc1, axis=0, keepdims=True)
    q1 = jnp.sum(acc1 * acc1, axis=0, keepdims=True)
    s2 = jnp.sum(acc2, axis=0, keepdims=True)
    q2 = jnp.sum(acc2 * acc2, axis=0, keepdims=True)
    blk = jnp.concatenate(
        [s1, q1, s2, q2, jnp.zeros((4, o2), jnp.float32)], axis=0)

    @pl.when(pl.program_id(0) == 0)
    def _():
        st_ref[...] = jnp.zeros_like(st_ref)

    st_ref[...] += blk


def _p2_body(n_total, x1r_ref, x2r_ref, st_ref, g1_ref, b1_ref, g2_ref,
             b2_ref, wm_ref, wft_ref, wfb_ref, outr_ref, stf_ref):
    st = st_ref[...]
    inv_n = 1.0 / n_total
    m1 = st[0:1, :] * inv_n
    v1 = st[1:2, :] * inv_n - m1 * m1
    a1 = g1_ref[...] / jnp.sqrt(v1 + EPS)
    c1 = b1_ref[...] - a1 * m1
    m2 = st[2:3, :] * inv_n
    v2 = st[3:4, :] * inv_n - m2 * m2
    a2 = g2_ref[...] / jnp.sqrt(v2 + EPS)
    c2 = b2_ref[...] - a2 * m2

    x1 = x1r_ref[...] * a1 + c1
    x2 = x2r_ref[...] * a2 + c2
    y2 = jnp.dot(x1 + x2, wm_ref[...], preferred_element_type=jnp.float32)
    outr = (jnp.dot(x1, wft_ref[...], preferred_element_type=jnp.float32)
            + jnp.dot(y2, wfb_ref[...], preferred_element_type=jnp.float32))
    outr_ref[...] = outr

    s = jnp.sum(outr, axis=0, keepdims=True)
    qq = jnp.sum(outr * outr, axis=0, keepdims=True)
    o = outr.shape[1]
    blk = jnp.concatenate([s, qq, jnp.zeros((6, o), jnp.float32)], axis=0)

    @pl.when(pl.program_id(0) == 0)
    def _():
        stf_ref[...] = jnp.zeros_like(stf_ref)

    stf_ref[...] += blk


def _p3_body(n_total, outr_ref, stf_ref, gf_ref, bf_ref, out_ref):
    st = stf_ref[...]
    inv_n = 1.0 / n_total
    m = st[0:1, :] * inv_n
    v = st[1:2, :] * inv_n - m * m
    a = gf_ref[...] / jnp.sqrt(v + EPS)
    c = bf_ref[...] - a * m
    out_ref[...] = jnp.maximum(outr_ref[...] * a + c, 0.0)


def kernel(q_pts, s_pts, neighb_inds, x, stack_lengths_post, KP_mini, W_mini,
           gamma1, beta1, KP_mid, W_mid, gamma2, beta2, W_midmini, W_final,
           gamma_f, beta_f):
    n, c = x.shape
    h = neighb_inds.shape[1]
    k1 = KP_mini.shape[0]
    k2 = KP_mid.shape[0]
    kk = k1 + k2
    o2 = W_mini.shape[2]
    o = W_final.shape[1]

    inds_flat = neighb_inds.astype(jnp.int32).reshape(-1)
    feat, gx, gy, gz = _sc_gather(
        x, s_pts[:, 0], s_pts[:, 1], s_pts[:, 2], inds_flat)
    g3 = feat.reshape(n, h, c)
    nbx = gx.reshape(n, h)
    nby = gy.reshape(n, h)
    nbz = gz.reshape(n, h)

    kp_all = jnp.concatenate([KP_mini, KP_mid], axis=0)

    b = 200
    grid = (n // b,)
    x1r, x2r, st = pl.pallas_call(
        functools.partial(_p1_body, k1, k2, c, o2),
        grid=grid,
        in_specs=[
            pl.BlockSpec((b, h, c), lambda i: (i, 0, 0)),
            pl.BlockSpec((b, h), lambda i: (i, 0)),
            pl.BlockSpec((b, h), lambda i: (i, 0)),
            pl.BlockSpec((b, h), lambda i: (i, 0)),
            pl.BlockSpec((b, 3), lambda i: (i, 0)),
            pl.BlockSpec((kk, 3), lambda i: (0, 0)),
            pl.BlockSpec((k1, c, o2), lambda i: (0, 0, 0)),
            pl.BlockSpec((k2, c, o2), lambda i: (0, 0, 0)),
        ],
        out_specs=[
            pl.BlockSpec((b, o2), lambda i: (i, 0)),
            pl.BlockSpec((b, o2), lambda i: (i, 0)),
            pl.BlockSpec((8, o2), lambda i: (0, 0)),
        ],
        out_shape=[
            jax.ShapeDtypeStruct((n, o2), jnp.float32),
            jax.ShapeDtypeStruct((n, o2), jnp.float32),
            jax.ShapeDtypeStruct((8, o2), jnp.float32),
        ],
        scratch_shapes=[
            pltpu.VMEM((b, kk, h), jnp.float32),
            pltpu.VMEM((b, kk, c), jnp.float32),
        ],
    )(g3, nbx, nby, nbz, q_pts, kp_all, W_mini, W_mid)

    wf_top = W_final[:o2, :]
    wf_bot = W_final[o2:, :]
    outr, stf = pl.pallas_call(
        functools.partial(_p2_body, n),
        grid=grid,
        in_specs=[
            pl.BlockSpec((b, o2), lambda i: (i, 0)),
            pl.BlockSpec((b, o2), lambda i: (i, 0)),
            pl.BlockSpec((8, o2), lambda i: (0, 0)),
            pl.BlockSpec((1, o2), lambda i: (0, 0)),
            pl.BlockSpec((1, o2), lambda i: (0, 0)),
            pl.BlockSpec((1, o2), lambda i: (0, 0)),
            pl.BlockSpec((1, o2), lambda i: (0, 0)),
            pl.BlockSpec((o2, o2), lambda i: (0, 0)),
            pl.BlockSpec((o2, o), lambda i: (0, 0)),
            pl.BlockSpec((o2, o), lambda i: (0, 0)),
        ],
        out_specs=[
            pl.BlockSpec((b, o), lambda i: (i, 0)),
            pl.BlockSpec((8, o), lambda i: (0, 0)),
        ],
        out_shape=[
            jax.ShapeDtypeStruct((n, o), jnp.float32),
            jax.ShapeDtypeStruct((8, o), jnp.float32),
        ],
    )(x1r, x2r, st, gamma1.reshape(1, o2), beta1.reshape(1, o2),
      gamma2.reshape(1, o2), beta2.reshape(1, o2), W_midmini, wf_top, wf_bot)

    out = pl.pallas_call(
        functools.partial(_p3_body, n),
        grid=grid,
        in_specs=[
            pl.BlockSpec((b, o), lambda i: (i, 0)),
            pl.BlockSpec((8, o), lambda i: (0, 0)),
            pl.BlockSpec((1, o), lambda i: (0, 0)),
            pl.BlockSpec((1, o), lambda i: (0, 0)),
        ],
        out_specs=pl.BlockSpec((b, o), lambda i: (i, 0)),
        out_shape=jax.ShapeDtypeStruct((n, o), jnp.float32),
    )(outr, stf, gamma_f.reshape(1, o), beta_f.reshape(1, o))

    return out


# speedup vs baseline: 5.7288x; 3.3958x over previous
import functools

import jax
import jax.numpy as jnp
from jax import lax
from jax.experimental import pallas as pl
from jax.experimental.pallas import tpu as pltpu
from jax.experimental.pallas import tpu_sc as plsc

KP_EXTENT = 2.0
EPS = 1e-5


def _sc_gather(x, sx, sy, sz, inds_flat, chunk=400):
    n, c = x.shape
    e = inds_flat.shape[0]
    info = plsc.get_sparse_core_info()
    nw = info.num_cores * info.num_subcores
    per_w = e // nw
    assert e % nw == 0 and per_w % chunk == 0 and chunk % 16 == 0
    n_ch = per_w // chunk
    n_sub = chunk // 16
    mesh = plsc.VectorSubcoreMesh(core_axis_name="c", subcore_axis_name="s")

    @functools.partial(
        pl.kernel,
        mesh=mesh,
        out_type=(
            jax.ShapeDtypeStruct((e, c), jnp.float32),
            jax.ShapeDtypeStruct((e,), jnp.float32),
            jax.ShapeDtypeStruct((e,), jnp.float32),
            jax.ShapeDtypeStruct((e,), jnp.float32),
        ),
        scratch_types=[
            pltpu.VMEM((chunk,), jnp.int32),
            pltpu.VMEM((chunk, c), jnp.float32),
            pltpu.VMEM((n,), jnp.float32),
            pltpu.VMEM((n,), jnp.float32),
            pltpu.VMEM((n,), jnp.float32),
            pltpu.VMEM((chunk,), jnp.float32),
            pltpu.VMEM((chunk,), jnp.float32),
            pltpu.VMEM((chunk,), jnp.float32),
            pltpu.SemaphoreType.DMA,
        ],
        compiler_params=pltpu.CompilerParams(needs_layout_passes=False),
    )
    def gather_k(x_hbm, sx_hbm, sy_hbm, sz_hbm, idx_hbm,
                 feat_hbm, ox_hbm, oy_hbm, oz_hbm,
                 idx_v, rows_v, sxv, syv, szv, gxv, gyv, gzv, sem):
        wid = lax.axis_index("s") * info.num_cores + lax.axis_index("c")
        base0 = wid * per_w
        pltpu.sync_copy(sx_hbm, sxv)
        pltpu.sync_copy(sy_hbm, syv)
        pltpu.sync_copy(sz_hbm, szv)

        def chunk_body(i, carry):
            base = base0 + i * chunk
            pltpu.sync_copy(idx_hbm.at[pl.ds(base, chunk)], idx_v)
            pltpu.async_copy(x_hbm.at[idx_v], rows_v, sem).wait()

            def sub_body(j, carry2):
                off = j * 16
                iv = idx_v[pl.ds(off, 16)]
                gxv[pl.ds(off, 16)] = plsc.load_gather(sxv, [iv])
                gyv[pl.ds(off, 16)] = plsc.load_gather(syv, [iv])
                gzv[pl.ds(off, 16)] = plsc.load_gather(szv, [iv])
                return carry2

            lax.fori_loop(0, n_sub, sub_body, 0)
            pltpu.sync_copy(rows_v, feat_hbm.at[pl.ds(base, chunk)])
            pltpu.sync_copy(gxv, ox_hbm.at[pl.ds(base, chunk)])
            pltpu.sync_copy(gyv, oy_hbm.at[pl.ds(base, chunk)])
            pltpu.sync_copy(gzv, oz_hbm.at[pl.ds(base, chunk)])
            return carry

        lax.fori_loop(0, n_ch, chunk_body, 0)

    return gather_k(x, sx, sy, sz, inds_flat)


def _p1_body(k1, k2, c, o2, g_ref, nbx_ref, nby_ref, nbz_ref, q_ref, kp_ref,
             wm_ref, wd_ref, x1r_ref, x2r_ref, st_ref, w3_ref, wf_ref):
    b = q_ref.shape[0]
    h = nbx_ref.shape[1]
    kk = k1 + k2
    nbx = nbx_ref[...] - q_ref[:, 0:1]
    nby = nby_ref[...] - q_ref[:, 1:2]
    nbz = nbz_ref[...] - q_ref[:, 2:3]

    inv_ext = 1.0 / KP_EXTENT
    wks = []
    for k in range(kk):
        dx = nbx - kp_ref[k:k + 1, 0:1]
        dy = nby - kp_ref[k:k + 1, 1:2]
        dz = nbz - kp_ref[k:k + 1, 2:3]
        sqk = dx * dx + dy * dy + dz * dz
        wk = jnp.maximum(1.0 - jnp.sqrt(sqk + 1e-12) * inv_ext, 0.0)
        wks.append(wk)
    w3_ref[...] = jnp.stack(wks, axis=1)

    for bb in range(b):
        w_b = w3_ref[bb]
        f_b = g_ref[bb]
        wf_ref[bb] = jnp.dot(w_b, f_b, preferred_element_type=jnp.float32)

    wf = wf_ref[...]
    acc1 = jnp.zeros((b, o2), jnp.float32)
    acc2 = jnp.zeros((b, o2), jnp.float32)
    for k in range(kk):
        wfk = wf[:, k, :]
        if k < k1:
            acc1 = acc1 + jnp.dot(wfk, wm_ref[k],
                                  preferred_element_type=jnp.float32)
        else:
            acc2 = acc2 + jnp.dot(wfk, wd_ref[k - k1],
                                  preferred_element_type=jnp.float32)

    x1r_ref[...] = acc1
    x2r_ref[...] = acc2

    s1 = jnp.sum(acc1, axis=0, keepdims=True)
    q1 = jnp.sum(acc1 * acc1, axis=0, keepdims=True)
    s2 = jnp.sum(acc2, axis=0, keepdims=True)
    q2 = jnp.sum(acc2 * acc2, axis=0, keepdims=True)
    blk = jnp.concatenate(
        [s1, q1, s2, q2, jnp.zeros((4, o2), jnp.float32)], axis=0)

    @pl.when(pl.program_id(0) == 0)
    def _():
        st_ref[...] = jnp.zeros_like(st_ref)

    st_ref[...] += blk


def _p2_body(n_total, x1r_ref, x2r_ref, st_ref, g1_ref, b1_ref, g2_ref,
             b2_ref, wm_ref, wft_ref, wfb_ref, outr_ref, stf_ref):
    st = st_ref[...]
    inv_n = 1.0 / n_total
    m1 = st[0:1, :] * inv_n
    v1 = st[1:2, :] * inv_n - m1 * m1
    a1 = g1_ref[...] / jnp.sqrt(v1 + EPS)
    c1 = b1_ref[...] - a1 * m1
    m2 = st[2:3, :] * inv_n
    v2 = st[3:4, :] * inv_n - m2 * m2
    a2 = g2_ref[...] / jnp.sqrt(v2 + EPS)
    c2 = b2_ref[...] - a2 * m2

    x1 = x1r_ref[...] * a1 + c1
    x2 = x2r_ref[...] * a2 + c2
    y2 = jnp.dot(x1 + x2, wm_ref[...], preferred_element_type=jnp.float32)
    outr = (jnp.dot(x1, wft_ref[...], preferred_element_type=jnp.float32)
            + jnp.dot(y2, wfb_ref[...], preferred_element_type=jnp.float32))
    outr_ref[...] = outr

    s = jnp.sum(outr, axis=0, keepdims=True)
    qq = jnp.sum(outr * outr, axis=0, keepdims=True)
    o = outr.shape[1]
    blk = jnp.concatenate([s, qq, jnp.zeros((6, o), jnp.float32)], axis=0)

    @pl.when(pl.program_id(0) == 0)
    def _():
        stf_ref[...] = jnp.zeros_like(stf_ref)

    stf_ref[...] += blk


def _p3_body(n_total, outr_ref, stf_ref, gf_ref, bf_ref, out_ref):
    st = stf_ref[...]
    inv_n = 1.0 / n_total
    m = st[0:1, :] * inv_n
    v = st[1:2, :] * inv_n - m * m
    a = gf_ref[...] / jnp.sqrt(v + EPS)
    c = bf_ref[...] - a * m
    out_ref[...] = jnp.maximum(outr_ref[...] * a + c, 0.0)


def kernel(q_pts, s_pts, neighb_inds, x, stack_lengths_post, KP_mini, W_mini,
           gamma1, beta1, KP_mid, W_mid, gamma2, beta2, W_midmini, W_final,
           gamma_f, beta_f):
    n, c = x.shape
    h = neighb_inds.shape[1]
    k1 = KP_mini.shape[0]
    k2 = KP_mid.shape[0]
    kk = k1 + k2
    o2 = W_mini.shape[2]
    o = W_final.shape[1]

    inds_flat = neighb_inds.astype(jnp.int32).reshape(-1)
    feat, gx, gy, gz = _sc_gather(
        x, s_pts[:, 0], s_pts[:, 1], s_pts[:, 2], inds_flat)
    g3 = feat.reshape(n, h, c)
    nbx = gx.reshape(n, h)
    nby = gy.reshape(n, h)
    nbz = gz.reshape(n, h)

    kp_all = jnp.concatenate([KP_mini, KP_mid], axis=0)

    b = 200
    grid = (n // b,)
    x1r, x2r, st = pl.pallas_call(
        functools.partial(_p1_body, k1, k2, c, o2),
        grid=grid,
        in_specs=[
            pl.BlockSpec((b, h, c), lambda i: (i, 0, 0)),
            pl.BlockSpec((b, h), lambda i: (i, 0)),
            pl.BlockSpec((b, h), lambda i: (i, 0)),
            pl.BlockSpec((b, h), lambda i: (i, 0)),
            pl.BlockSpec((b, 3), lambda i: (i, 0)),
            pl.BlockSpec((kk, 3), lambda i: (0, 0)),
            pl.BlockSpec((k1, c, o2), lambda i: (0, 0, 0)),
            pl.BlockSpec((k2, c, o2), lambda i: (0, 0, 0)),
        ],
        out_specs=[
            pl.BlockSpec((b, o2), lambda i: (i, 0)),
            pl.BlockSpec((b, o2), lambda i: (i, 0)),
            pl.BlockSpec((8, o2), lambda i: (0, 0)),
        ],
        out_shape=[
            jax.ShapeDtypeStruct((n, o2), jnp.float32),
            jax.ShapeDtypeStruct((n, o2), jnp.float32),
            jax.ShapeDtypeStruct((8, o2), jnp.float32),
        ],
        scratch_shapes=[
            pltpu.VMEM((b, kk, h), jnp.float32),
            pltpu.VMEM((b, kk, c), jnp.float32),
        ],
    )(g3, nbx, nby, nbz, q_pts, kp_all, W_mini, W_mid)

    wf_top = W_final[:o2, :]
    wf_bot = W_final[o2:, :]
    outr, stf = pl.pallas_call(
        functools.partial(_p2_body, n),
        grid=grid,
        in_specs=[
            pl.BlockSpec((b, o2), lambda i: (i, 0)),
            pl.BlockSpec((b, o2), lambda i: (i, 0)),
            pl.BlockSpec((8, o2), lambda i: (0, 0)),
            pl.BlockSpec((1, o2), lambda i: (0, 0)),
            pl.BlockSpec((1, o2), lambda i: (0, 0)),
            pl.BlockSpec((1, o2), lambda i: (0, 0)),
            pl.BlockSpec((1, o2), lambda i: (0, 0)),
            pl.BlockSpec((o2, o2), lambda i: (0, 0)),
            pl.BlockSpec((o2, o), lambda i: (0, 0)),
            pl.BlockSpec((o2, o), lambda i: (0, 0)),
        ],
        out_specs=[
            pl.BlockSpec((b, o), lambda i: (i, 0)),
            pl.BlockSpec((8, o), lambda i: (0, 0)),
        ],
        out_shape=[
            jax.ShapeDtypeStruct((n, o), jnp.float32),
            jax.ShapeDtypeStruct((8, o), jnp.float32),
        ],
    )(x1r, x2r, st, gamma1.reshape(1, o2), beta1.reshape(1, o2),
      gamma2.reshape(1, o2), beta2.reshape(1, o2), W_midmini, wf_top, wf_bot)

    out = pl.pallas_call(
        functools.partial(_p3_body, n),
        grid=grid,
        in_specs=[
            pl.BlockSpec((b, o), lambda i: (i, 0)),
            pl.BlockSpec((8, o), lambda i: (0, 0)),
            pl.BlockSpec((1, o), lambda i: (0, 0)),
            pl.BlockSpec((1, o), lambda i: (0, 0)),
        ],
        out_specs=pl.BlockSpec((b, o), lambda i: (i, 0)),
        out_shape=jax.ShapeDtypeStruct((n, o), jnp.float32),
    )(outr, stf, gamma_f.reshape(1, o), beta_f.reshape(1, o))

    return out
